# baseline (speedup 1.0000x reference)
import functools

import jax
import jax.numpy as jnp
from jax import lax
from jax.experimental import pallas as pl
from jax.experimental.pallas import tpu as pltpu
from jax.experimental.pallas import tpu_sc as plsc

B, M, L = 1024, 4, 200
V, H, P = 100000, 64, 26
SEG = B * M
CHUNKS = ((0, 128), (128, 72))
PP = 32
LANE = 16

_ROWS = 4096


def _proj_body(a_ref, f_ref, w_ref, b_ref, z_ref):
    wa = w_ref[:H, :]
    wf = w_ref[H:, :]
    dn = (((0,), (0,)), ((), ()))
    xa = lax.dot_general(a_ref[...], wa, dn,
                         preferred_element_type=jnp.float32) + b_ref[...]
    xf = lax.dot_general(f_ref[...], wf, dn,
                         preferred_element_type=jnp.float32)
    z_ref[:, :H] = xa.astype(jnp.bfloat16)
    z_ref[:, H:] = xf.astype(jnp.bfloat16)


def _project_tables(atom_emb, fp_emb, W_in, b_in):
    z = pl.pallas_call(
        _proj_body,
        grid=(pl.cdiv(V, _ROWS),),
        in_specs=[
            pl.BlockSpec((H, _ROWS), lambda i: (0, i)),
            pl.BlockSpec((H, _ROWS), lambda i: (0, i)),
            pl.BlockSpec((2 * H, H), lambda i: (0, 0)),
            pl.BlockSpec((1, H), lambda i: (0, 0)),
        ],
        out_specs=pl.BlockSpec((_ROWS, 2 * H), lambda i: (i, 0)),
        out_shape=jax.ShapeDtypeStruct((V, 2 * H), jnp.bfloat16),
    )(atom_emb.T, fp_emb.T, W_in, b_in.reshape(1, H))
    return z.reshape(2 * V, H)


def _sc_gather_sum(z, idx_a, idx_f):
    info = plsc.get_sparse_core_info()
    NC, NS = info.num_cores, info.num_subcores
    NW = NC * NS
    spw = SEG // NW
    bpw = B // NW

    @functools.partial(
        pl.kernel,
        mesh=plsc.VectorSubcoreMesh(core_axis_name="c", subcore_axis_name="s"),
        out_type=jax.ShapeDtypeStruct((SEG, H), jnp.float32),
        compiler_params=pltpu.CompilerParams(use_tc_tiling_on_sc=False),
        scratch_types=[
            pltpu.VMEM((bpw, M, L), jnp.int32),
            pltpu.VMEM((bpw, M, L), jnp.int32),
            pltpu.VMEM((2, L, H), jnp.bfloat16),
            pltpu.VMEM((2, L, H), jnp.bfloat16),
            pltpu.VMEM((spw, H), jnp.float32),
            pltpu.SemaphoreType.DMA,
            pltpu.SemaphoreType.DMA,
        ],
    )
    def k(z_hbm, ia_hbm, if_hbm, out_hbm,
          ia_v, if_v, pa_v, pf_v, out_v, sem0, sem1):
        sems = (sem0, sem1)
        wid = lax.axis_index("s") * NC + lax.axis_index("c")
        base = wid * spw

        pltpu.sync_copy(ia_hbm.at[pl.ds(wid * bpw, bpw)], ia_v)
        pltpu.sync_copy(if_hbm.at[pl.ds(wid * bpw, bpw)], if_v)

        def copies(s, b):
            bb = s // M
            m = s % M
            cs = []
            for off, sz in CHUNKS:
                cs.append(pltpu.make_async_copy(
                    z_hbm.at[ia_v.at[bb, m, pl.ds(off, sz)]],
                    pa_v.at[b, pl.ds(off, sz)], sems[b]))
                cs.append(pltpu.make_async_copy(
                    z_hbm.at[if_v.at[bb, m, pl.ds(off, sz)]],
                    pf_v.at[b, pl.ds(off, sz)], sems[b]))
            return cs

        def issue(s, b):
            for cp in copies(s, b):
                cp.start()

        def drain(s, b):
            for cp in copies(s, b):
                cp.wait()

        def compute(s, b):
            def body(i, acc):
                nxt = []
                for j in range(H // (2 * LANE)):
                    v = (pa_v[b, i, pl.ds(j * 2 * LANE, 2 * LANE)]
                         + pf_v[b, i, pl.ds(j * 2 * LANE, 2 * LANE)])
                    r = jnp.maximum(v, jnp.bfloat16(0.0)).astype(jnp.float32)
                    nxt.append(acc[j] + r)
                return tuple(nxt)

            zero = jnp.zeros((2 * LANE,), jnp.float32)
            acc = lax.fori_loop(0, L, body, (zero,) * (H // (2 * LANE)))
            for j in range(H // (2 * LANE)):
                out_v[s, pl.ds(j * 2 * LANE, 2 * LANE)] = acc[j]

        issue(0, 0)

        def step(t, carry):
            s0 = 2 * t
            drain(s0, 0)
            issue(s0 + 1, 1)
            compute(s0, 0)
            s1 = s0 + 1
            drain(s1, 1)

            @pl.when(t < spw // 2 - 1)
            def _():
                issue(s1 + 1, 0)

            compute(s1, 1)
            return carry

        lax.fori_loop(0, spw // 2, step, 0)
        pltpu.sync_copy(out_v, out_hbm.at[pl.ds(base, spw)])

    return k(z, idx_a, idx_f)


def _tail_body(s_ref, pv_ref, r_ref, gh_ref, bh_ref, gp_ref, bp_ref,
               w1a_ref, w1b_ref, b1_ref, w2_ref, b2_ref, y_ref):
    r = r_ref[...]
    w = r / (jnp.sum(r, axis=1, keepdims=True) + 1e-8)
    wl = w * (1.0 / L)

    mix_h = jnp.zeros((B, H), jnp.float32)
    mix_p = jnp.zeros((B, PP), jnp.float32)
    for m in range(M):
        sm = s_ref[:, m * H:(m + 1) * H]
        pv = jnp.nan_to_num(pv_ref[:, m * PP:(m + 1) * PP],
                            nan=0.0, posinf=1000.0, neginf=-1000.0)
        mix_h = mix_h + wl[:, m:m + 1] * sm
        mix_p = mix_p + w[:, m:m + 1] * pv

    n = float(H + P)
    mu = (jnp.sum(mix_h, axis=1, keepdims=True)
          + jnp.sum(mix_p, axis=1, keepdims=True)) * (1.0 / n)
    ch = mix_h - mu
    cp = mix_p - mu
    sq = (jnp.sum(ch * ch, axis=1, keepdims=True)
          + jnp.sum(cp * cp, axis=1, keepdims=True)
          - float(PP - P) * mu * mu)
    inv = lax.rsqrt(sq * (1.0 / n) + 1e-5)
    znh = ch * inv * gh_ref[...] + bh_ref[...]
    znp = cp * inv * gp_ref[...] + bp_ref[...]

    h = jnp.maximum(
        jnp.dot(znh, w1a_ref[...], preferred_element_type=jnp.float32)
        + jnp.dot(znp, w1b_ref[...], preferred_element_type=jnp.float32)
        + b1_ref[...], 0.0)
    y = jnp.dot(h, w2_ref[...], preferred_element_type=jnp.float32) + b2_ref[...]
    y_ref[...] = jnp.nan_to_num(y)


def _tail(S, pv, ratios, ln_g, ln_b, W1, b1, W2, b2):
    gh = ln_g[:H].reshape(1, H)
    bh = ln_b[:H].reshape(1, H)
    gp = jnp.pad(ln_g[H:], (0, PP - P)).reshape(1, PP)
    bp = jnp.pad(ln_b[H:], (0, PP - P)).reshape(1, PP)
    w1a = W1[:H]
    w1b = jnp.pad(W1[H:], ((0, PP - P), (0, 0)))
    pvp = jnp.pad(pv, ((0, 0), (0, 0), (0, PP - P))).reshape(B, M * PP)
    return pl.pallas_call(
        _tail_body,
        out_shape=jax.ShapeDtypeStruct((B, 1), jnp.float32),
    )(S.reshape(B, M * H), pvp, ratios, gh, bh, gp, bp,
      w1a, w1b, b1.reshape(1, H), W2, b2.reshape(1, 1))


def kernel(atom_features, fingerprints, physicochemical, ratios,
           atom_emb, fp_emb, W_in, b_in, ln_g, ln_b, W1, b1, W2, b2):
    z = _project_tables(atom_emb, fp_emb, W_in, b_in)
    ia = atom_features.astype(jnp.int32) * 2
    fi = fingerprints.astype(jnp.int32) * 2 + 1
    S = _sc_gather_sum(z, ia, fi)
    return _tail(S, physicochemical, ratios, ln_g, ln_b, W1, b1, W2, b2)


# speedup vs baseline: 1.2277x; 1.2277x over previous
import functools

import jax
import jax.numpy as jnp
from jax import lax
from jax.experimental import pallas as pl
from jax.experimental.pallas import tpu as pltpu
from jax.experimental.pallas import tpu_sc as plsc

B, M, L = 1024, 4, 200
V, H, P = 100000, 64, 26
SEG = B * M
CHUNKS = ((0, 128), (128, 72))
PP = 32
LANE = 16

_ROWS = 4096


def _pack_bf16(x):
    u = lax.bitcast_convert_type(x, jnp.uint32)
    r = (u + jnp.uint32(0x7FFF) + ((u >> 16) & jnp.uint32(1))) >> 16
    return r[:, :H // 2] | (r[:, H // 2:] << 16)


def _proj_body(a_ref, f_ref, w_ref, b_ref, z_ref):
    wa = w_ref[:H, :]
    wf = w_ref[H:, :]
    dn = (((0,), (0,)), ((), ()))
    xa = lax.dot_general(a_ref[...], wa, dn,
                         preferred_element_type=jnp.float32) + b_ref[...]
    xf = lax.dot_general(f_ref[...], wf, dn,
                         preferred_element_type=jnp.float32)
    z_ref[:, :H // 2] = _pack_bf16(xa)
    z_ref[:, H // 2:H] = _pack_bf16(xf)


def _project_tables(atom_emb, fp_emb, W_in, b_in):
    z = pl.pallas_call(
        _proj_body,
        grid=(pl.cdiv(V, _ROWS),),
        in_specs=[
            pl.BlockSpec((H, _ROWS), lambda i: (0, i)),
            pl.BlockSpec((H, _ROWS), lambda i: (0, i)),
            pl.BlockSpec((2 * H, H), lambda i: (0, 0)),
            pl.BlockSpec((1, H), lambda i: (0, 0)),
        ],
        out_specs=pl.BlockSpec((_ROWS, 2 * H), lambda i: (i, 0)),
        out_shape=jax.ShapeDtypeStruct((V, 2 * H), jnp.uint32),
    )(atom_emb.T, fp_emb.T, W_in, b_in.reshape(1, H))
    return z.reshape(4 * V, H // 2)


def _sc_gather_sum(z, idx_a, idx_f):
    info = plsc.get_sparse_core_info()
    NC, NS = info.num_cores, info.num_subcores
    NW = NC * NS
    spw = SEG // NW
    bpw = B // NW

    @functools.partial(
        pl.kernel,
        mesh=plsc.VectorSubcoreMesh(core_axis_name="c", subcore_axis_name="s"),
        out_type=jax.ShapeDtypeStruct((SEG, H), jnp.float32),
        compiler_params=pltpu.CompilerParams(use_tc_tiling_on_sc=False),
        scratch_types=[
            pltpu.VMEM((bpw, M, L), jnp.int32),
            pltpu.VMEM((bpw, M, L), jnp.int32),
            pltpu.VMEM((2, L, H // 2), jnp.uint32),
            pltpu.VMEM((2, L, H // 2), jnp.uint32),
            pltpu.VMEM((spw, H), jnp.float32),
            pltpu.SemaphoreType.DMA,
            pltpu.SemaphoreType.DMA,
        ],
    )
    def k(z_hbm, ia_hbm, if_hbm, out_hbm,
          ia_v, if_v, pa_v, pf_v, out_v, sem0, sem1):
        sems = (sem0, sem1)
        wid = lax.axis_index("s") * NC + lax.axis_index("c")
        base = wid * spw

        pltpu.sync_copy(ia_hbm.at[pl.ds(wid * bpw, bpw)], ia_v)
        pltpu.sync_copy(if_hbm.at[pl.ds(wid * bpw, bpw)], if_v)

        def copies(s, b):
            bb = s // M
            m = s % M
            cs = []
            for off, sz in CHUNKS:
                cs.append(pltpu.make_async_copy(
                    z_hbm.at[ia_v.at[bb, m, pl.ds(off, sz)]],
                    pa_v.at[b, pl.ds(off, sz)], sems[b]))
                cs.append(pltpu.make_async_copy(
                    z_hbm.at[if_v.at[bb, m, pl.ds(off, sz)]],
                    pf_v.at[b, pl.ds(off, sz)], sems[b]))
            return cs

        def issue(s, b):
            for cp in copies(s, b):
                cp.start()

        def drain(s, b):
            for cp in copies(s, b):
                cp.wait()

        def compute(s, b):
            def unpack(w):
                lo = lax.bitcast_convert_type(w << 16, jnp.float32)
                hi = lax.bitcast_convert_type(w & jnp.uint32(0xFFFF0000),
                                              jnp.float32)
                return lo, hi

            def body(i, acc):
                nxt = []
                for j in range(H // (2 * LANE)):
                    a_lo, a_hi = unpack(pa_v[b, i, pl.ds(j * LANE, LANE)])
                    f_lo, f_hi = unpack(pf_v[b, i, pl.ds(j * LANE, LANE)])
                    nxt.append(acc[2 * j] + jnp.maximum(a_lo + f_lo, 0.0))
                    nxt.append(acc[2 * j + 1] + jnp.maximum(a_hi + f_hi, 0.0))
                return tuple(nxt)

            zero = jnp.zeros((LANE,), jnp.float32)
            acc = lax.fori_loop(0, L, body, (zero,) * (H // LANE))
            out_v[s, pl.ds(0 * LANE, LANE)] = acc[0]
            out_v[s, pl.ds(1 * LANE, LANE)] = acc[2]
            out_v[s, pl.ds(2 * LANE, LANE)] = acc[1]
            out_v[s, pl.ds(3 * LANE, LANE)] = acc[3]

        issue(0, 0)

        def step(t, carry):
            s0 = 2 * t
            drain(s0, 0)
            issue(s0 + 1, 1)
            compute(s0, 0)
            s1 = s0 + 1
            drain(s1, 1)

            @pl.when(t < spw // 2 - 1)
            def _():
                issue(s1 + 1, 0)

            compute(s1, 1)
            return carry

        lax.fori_loop(0, spw // 2, step, 0)
        pltpu.sync_copy(out_v, out_hbm.at[pl.ds(base, spw)])

    return k(z, idx_a, idx_f)


def _tail_body(s_ref, pv_ref, r_ref, gh_ref, bh_ref, gp_ref, bp_ref,
               w1a_ref, w1b_ref, b1_ref, w2_ref, b2_ref, y_ref):
    r = r_ref[...]
    w = r / (jnp.sum(r, axis=1, keepdims=True) + 1e-8)
    wl = w * (1.0 / L)

    mix_h = jnp.zeros((B, H), jnp.float32)
    mix_p = jnp.zeros((B, PP), jnp.float32)
    for m in range(M):
        sm = s_ref[:, m * H:(m + 1) * H]
        pv = jnp.nan_to_num(pv_ref[:, m * PP:(m + 1) * PP],
                            nan=0.0, posinf=1000.0, neginf=-1000.0)
        mix_h = mix_h + wl[:, m:m + 1] * sm
        mix_p = mix_p + w[:, m:m + 1] * pv

    n = float(H + P)
    mu = (jnp.sum(mix_h, axis=1, keepdims=True)
          + jnp.sum(mix_p, axis=1, keepdims=True)) * (1.0 / n)
    ch = mix_h - mu
    cp = mix_p - mu
    sq = (jnp.sum(ch * ch, axis=1, keepdims=True)
          + jnp.sum(cp * cp, axis=1, keepdims=True)
          - float(PP - P) * mu * mu)
    inv = lax.rsqrt(sq * (1.0 / n) + 1e-5)
    znh = ch * inv * gh_ref[...] + bh_ref[...]
    znp = cp * inv * gp_ref[...] + bp_ref[...]

    h = jnp.maximum(
        jnp.dot(znh, w1a_ref[...], preferred_element_type=jnp.float32)
        + jnp.dot(znp, w1b_ref[...], preferred_element_type=jnp.float32)
        + b1_ref[...], 0.0)
    y = jnp.dot(h, w2_ref[...], preferred_element_type=jnp.float32) + b2_ref[...]
    y_ref[...] = jnp.nan_to_num(y)


def _tail(S, pv, ratios, ln_g, ln_b, W1, b1, W2, b2):
    gh = ln_g[:H].reshape(1, H)
    bh = ln_b[:H].reshape(1, H)
    gp = jnp.pad(ln_g[H:], (0, PP - P)).reshape(1, PP)
    bp = jnp.pad(ln_b[H:], (0, PP - P)).reshape(1, PP)
    w1a = W1[:H]
    w1b = jnp.pad(W1[H:], ((0, PP - P), (0, 0)))
    pvp = jnp.pad(pv, ((0, 0), (0, 0), (0, PP - P))).reshape(B, M * PP)
    return pl.pallas_call(
        _tail_body,
        out_shape=jax.ShapeDtypeStruct((B, 1), jnp.float32),
    )(S.reshape(B, M * H), pvp, ratios, gh, bh, gp, bp,
      w1a, w1b, b1.reshape(1, H), W2, b2.reshape(1, 1))


def kernel(atom_features, fingerprints, physicochemical, ratios,
           atom_emb, fp_emb, W_in, b_in, ln_g, ln_b, W1, b1, W2, b2):
    z = _project_tables(atom_emb, fp_emb, W_in, b_in)
    ia = atom_features.astype(jnp.int32) * 4
    fi = fingerprints.astype(jnp.int32) * 4 + 1
    S = _sc_gather_sum(z, ia, fi)
    return _tail(S, physicochemical, ratios, ln_g, ln_b, W1, b1, W2, b2)


# speedup vs baseline: 1.2306x; 1.0024x over previous
import functools

import jax
import jax.numpy as jnp
from jax import lax
from jax.experimental import pallas as pl
from jax.experimental.pallas import tpu as pltpu
from jax.experimental.pallas import tpu_sc as plsc

B, M, L = 1024, 4, 200
V, H, P = 100000, 64, 26
SEG = B * M
CHUNKS = ((0, 128), (128, 72))
PP = 32
LANE = 16

_ROWS = 4096


def _pack_bf16(x):
    u = lax.bitcast_convert_type(x, jnp.uint32)
    r = (u + jnp.uint32(0x7FFF) + ((u >> 16) & jnp.uint32(1))) >> 16
    return r[:, :H // 2] | (r[:, H // 2:] << 16)


def _proj_body(a_ref, f_ref, w_ref, b_ref, z_ref):
    wa = w_ref[:H, :]
    wf = w_ref[H:, :]
    dn = (((0,), (0,)), ((), ()))
    xa = lax.dot_general(a_ref[...], wa, dn,
                         preferred_element_type=jnp.float32) + b_ref[...]
    xf = lax.dot_general(f_ref[...], wf, dn,
                         preferred_element_type=jnp.float32)
    z_ref[:, :H // 2] = _pack_bf16(xa)
    z_ref[:, H // 2:H] = _pack_bf16(xf)


def _project_tables(atom_emb, fp_emb, W_in, b_in):
    z = pl.pallas_call(
        _proj_body,
        grid=(pl.cdiv(V, _ROWS),),
        in_specs=[
            pl.BlockSpec((H, _ROWS), lambda i: (0, i)),
            pl.BlockSpec((H, _ROWS), lambda i: (0, i)),
            pl.BlockSpec((2 * H, H), lambda i: (0, 0)),
            pl.BlockSpec((1, H), lambda i: (0, 0)),
        ],
        out_specs=pl.BlockSpec((_ROWS, 2 * H), lambda i: (i, 0)),
        out_shape=jax.ShapeDtypeStruct((V, 2 * H), jnp.uint32),
    )(atom_emb.T, fp_emb.T, W_in, b_in.reshape(1, H))
    return z.reshape(4 * V, H // 2)


def _sc_gather_sum(z, idx_a, idx_f):
    info = plsc.get_sparse_core_info()
    NC, NS = info.num_cores, info.num_subcores
    NW = NC * NS
    spw = SEG // NW
    bpw = B // NW

    @functools.partial(
        pl.kernel,
        mesh=plsc.VectorSubcoreMesh(core_axis_name="c", subcore_axis_name="s"),
        out_type=jax.ShapeDtypeStruct((SEG, H), jnp.float32),
        compiler_params=pltpu.CompilerParams(use_tc_tiling_on_sc=False),
        scratch_types=[
            pltpu.VMEM((bpw, M, L), jnp.int32),
            pltpu.VMEM((bpw, M, L), jnp.int32),
            pltpu.VMEM((2, L, H // 2), jnp.uint32),
            pltpu.VMEM((2, L, H // 2), jnp.uint32),
            pltpu.VMEM((spw, H), jnp.float32),
            pltpu.SemaphoreType.DMA,
            pltpu.SemaphoreType.DMA,
        ],
    )
    def k(z_hbm, ia_hbm, if_hbm, out_hbm,
          ia_v, if_v, pa_v, pf_v, out_v, sem0, sem1):
        sems = (sem0, sem1)
        wid = lax.axis_index("s") * NC + lax.axis_index("c")
        base = wid * spw

        pltpu.sync_copy(ia_hbm.at[pl.ds(wid * bpw, bpw)], ia_v)
        pltpu.sync_copy(if_hbm.at[pl.ds(wid * bpw, bpw)], if_v)

        def copies(s, b):
            bb = s // M
            m = s % M
            cs = []
            for off, sz in CHUNKS:
                cs.append(pltpu.make_async_copy(
                    z_hbm.at[ia_v.at[bb, m, pl.ds(off, sz)]],
                    pa_v.at[b, pl.ds(off, sz)], sems[b]))
                cs.append(pltpu.make_async_copy(
                    z_hbm.at[if_v.at[bb, m, pl.ds(off, sz)]],
                    pf_v.at[b, pl.ds(off, sz)], sems[b]))
            return cs

        def issue(s, b):
            for cp in copies(s, b):
                cp.start()

        def drain(s, b):
            for cp in copies(s, b):
                cp.wait()

        def compute(s, b):
            def unpack(w):
                lo = lax.bitcast_convert_type(w << 16, jnp.float32)
                hi = lax.bitcast_convert_type(w, jnp.float32)
                return lo, hi

            def one(i, acc):
                nxt = []
                for j in range(H // (2 * LANE)):
                    a_lo, a_hi = unpack(pa_v[b, i, pl.ds(j * LANE, LANE)])
                    f_lo, f_hi = unpack(pf_v[b, i, pl.ds(j * LANE, LANE)])
                    nxt.append(acc[2 * j] + jnp.maximum(a_lo + f_lo, 0.0))
                    nxt.append(acc[2 * j + 1] + jnp.maximum(a_hi + f_hi, 0.0))
                return tuple(nxt)

            def body(i, acc):
                return one(2 * i + 1, one(2 * i, acc))

            zero = jnp.zeros((LANE,), jnp.float32)
            acc = lax.fori_loop(0, L // 2, body, (zero,) * (H // LANE))
            out_v[s, pl.ds(0 * LANE, LANE)] = acc[0]
            out_v[s, pl.ds(1 * LANE, LANE)] = acc[2]
            out_v[s, pl.ds(2 * LANE, LANE)] = acc[1]
            out_v[s, pl.ds(3 * LANE, LANE)] = acc[3]

        issue(0, 0)

        def step(t, carry):
            s0 = 2 * t
            drain(s0, 0)
            issue(s0 + 1, 1)
            compute(s0, 0)
            s1 = s0 + 1
            drain(s1, 1)

            @pl.when(t < spw // 2 - 1)
            def _():
                issue(s1 + 1, 0)

            compute(s1, 1)
            return carry

        lax.fori_loop(0, spw // 2, step, 0)
        pltpu.sync_copy(out_v, out_hbm.at[pl.ds(base, spw)])

    return k(z, idx_a, idx_f)


def _tail_body(s_ref, pv_ref, r_ref, gh_ref, bh_ref, gp_ref, bp_ref,
               w1a_ref, w1b_ref, b1_ref, w2_ref, b2_ref, y_ref):
    r = r_ref[...]
    w = r / (jnp.sum(r, axis=1, keepdims=True) + 1e-8)
    wl = w * (1.0 / L)

    mix_h = jnp.zeros((B, H), jnp.float32)
    mix_p = jnp.zeros((B, PP), jnp.float32)
    for m in range(M):
        sm = s_ref[:, m * H:(m + 1) * H]
        pv = jnp.nan_to_num(pv_ref[:, m * PP:(m + 1) * PP],
                            nan=0.0, posinf=1000.0, neginf=-1000.0)
        mix_h = mix_h + wl[:, m:m + 1] * sm
        mix_p = mix_p + w[:, m:m + 1] * pv

    n = float(H + P)
    mu = (jnp.sum(mix_h, axis=1, keepdims=True)
          + jnp.sum(mix_p, axis=1, keepdims=True)) * (1.0 / n)
    ch = mix_h - mu
    cp = mix_p - mu
    sq = (jnp.sum(ch * ch, axis=1, keepdims=True)
          + jnp.sum(cp * cp, axis=1, keepdims=True)
          - float(PP - P) * mu * mu)
    inv = lax.rsqrt(sq * (1.0 / n) + 1e-5)
    znh = ch * inv * gh_ref[...] + bh_ref[...]
    znp = cp * inv * gp_ref[...] + bp_ref[...]

    h = jnp.maximum(
        jnp.dot(znh, w1a_ref[...], preferred_element_type=jnp.float32)
        + jnp.dot(znp, w1b_ref[...], preferred_element_type=jnp.float32)
        + b1_ref[...], 0.0)
    y = jnp.dot(h, w2_ref[...], preferred_element_type=jnp.float32) + b2_ref[...]
    y_ref[...] = jnp.nan_to_num(y)


def _tail(S, pv, ratios, ln_g, ln_b, W1, b1, W2, b2):
    gh = ln_g[:H].reshape(1, H)
    bh = ln_b[:H].reshape(1, H)
    gp = jnp.pad(ln_g[H:], (0, PP - P)).reshape(1, PP)
    bp = jnp.pad(ln_b[H:], (0, PP - P)).reshape(1, PP)
    w1a = W1[:H]
    w1b = jnp.pad(W1[H:], ((0, PP - P), (0, 0)))
    pvp = jnp.pad(pv, ((0, 0), (0, 0), (0, PP - P))).reshape(B, M * PP)
    return pl.pallas_call(
        _tail_body,
        out_shape=jax.ShapeDtypeStruct((B, 1), jnp.float32),
    )(S.reshape(B, M * H), pvp, ratios, gh, bh, gp, bp,
      w1a, w1b, b1.reshape(1, H), W2, b2.reshape(1, 1))


def kernel(atom_features, fingerprints, physicochemical, ratios,
           atom_emb, fp_emb, W_in, b_in, ln_g, ln_b, W1, b1, W2, b2):
    z = _project_tables(atom_emb, fp_emb, W_in, b_in)
    ia = atom_features.astype(jnp.int32) * 4
    fi = fingerprints.astype(jnp.int32) * 4 + 1
    S = _sc_gather_sum(z, ia, fi)
    return _tail(S, physicochemical, ratios, ln_g, ln_b, W1, b1, W2, b2)
